# baseline (speedup 1.0000x reference)
import functools

import jax
import jax.numpy as jnp
from jax import lax
from jax.experimental import pallas as pl
from jax.experimental.pallas import tpu as pltpu
from jax.experimental.pallas import tpu_sc as plsc

NC = 2
NS = 16
NW = NC * NS
L = 16

CH_GIN = 128
CH_GAT = 64
ZR = 128
F = 128


def _ceil_div(a, b):
    return (a + b - 1) // b


G = 16


def _make_segsum(n_rows, n_acc, nchunks_per_tile, ch):
    mesh = plsc.VectorSubcoreMesh(core_axis_name="c", subcore_axis_name="s")
    nzb = _ceil_div(n_acc, ch)
    nob = _ceil_div(n_acc, ZR)
    assert nchunks_per_tile % G == 0

    @functools.partial(
        pl.kernel,
        out_type=jax.ShapeDtypeStruct((NC, n_acc, F), jnp.float32),
        mesh=mesh,
        compiler_params=pltpu.CompilerParams(needs_layout_passes=False),
        scratch_types=[
            pltpu.VMEM((G, ch), jnp.int32),
            pltpu.VMEM((G, ch), jnp.int32),
            pltpu.VMEM((2, ch, F), jnp.float32),
            pltpu.VMEM_SHARED((n_rows, F), jnp.float32),
            pltpu.SemaphoreType.DMA,
            pltpu.SemaphoreType.DMA,
        ],
    )
    def segsum(x_hbm, eidx_hbm, out_hbm, sibuf, dibuf, rows, acc, gs0, gs1):
        core = lax.axis_index("c")
        sub = lax.axis_index("s")
        wid = sub * NC + core

        z16 = jnp.zeros((L,), jnp.float32)

        def zrow(i, _):
            for j in range(F // L):
                rows[0, i, pl.ds(j * L, L)] = z16
            return 0

        lax.fori_loop(0, ch, zrow, 0)

        def zblk(z, _):
            bz = sub + z * NS
            off = jnp.minimum(bz * ch, n_acc - ch)
            pltpu.sync_copy(rows.at[0], acc.at[pl.ds(off, ch), :])
            return 0

        nz = (nzb - sub + NS - 1) // NS
        lax.fori_loop(0, nz, zblk, 0)
        if n_rows > n_acc:

            @pl.when(sub == 0)
            def _():
                pltpu.sync_copy(
                    rows.at[0, pl.ds(0, n_rows - n_acc), :],
                    acc.at[pl.ds(n_acc, n_rows - n_acc), :],
                )

        plsc.subcore_barrier()

        def gather(kk, b, sem):
            return pltpu.make_async_copy(
                x_hbm.at[sibuf.at[kk]], rows.at[b], sem
            )

        def scat(kk, b):
            pltpu.sync_copy(rows.at[b], acc.at[dibuf.at[kk]], add=True)

        def grp(g, _):
            pltpu.sync_copy(eidx_hbm.at[0, wid, pl.ds(g * G, G)], sibuf)
            pltpu.sync_copy(eidx_hbm.at[1, wid, pl.ds(g * G, G)], dibuf)
            gather(0, 0, gs0).start()

            def pair(p, _):
                k0 = 2 * p
                k1 = k0 + 1
                gather(k1, 1, gs1).start()
                gather(k0, 0, gs0).wait()
                scat(k0, 0)

                @pl.when(k1 + 1 < G)
                def _():
                    gather(k1 + 1, 0, gs0).start()

                gather(k1, 1, gs1).wait()
                scat(k1, 1)
                return 0

            lax.fori_loop(0, G // 2, pair, 0)
            return 0

        lax.fori_loop(0, nchunks_per_tile // G, grp, 0)

        plsc.subcore_barrier()

        def oblk(z, _):
            bz = sub + z * NS
            off = jnp.minimum(bz * ZR, n_acc - ZR)
            pltpu.sync_copy(
                acc.at[pl.ds(off, ZR), :], out_hbm.at[core, pl.ds(off, ZR), :]
            )
            return 0

        no = (nob - sub + NS - 1) // NS
        lax.fori_loop(0, no, oblk, 0)

    return segsum


def _make_gat_edges(n_rows, n_acc, nchunks_per_tile, ch):
    mesh = plsc.VectorSubcoreMesh(core_axis_name="c", subcore_axis_name="s")
    nzb = _ceil_div(n_acc, ch)
    nob = _ceil_div(n_acc, ZR)
    assert nchunks_per_tile % G == 0

    @functools.partial(
        pl.kernel,
        out_type=(
            jax.ShapeDtypeStruct((NC, n_acc, F), jnp.float32),
            jax.ShapeDtypeStruct((NC * n_acc,), jnp.float32),
        ),
        mesh=mesh,
        compiler_params=pltpu.CompilerParams(needs_layout_passes=False),
        scratch_types=[
            pltpu.VMEM((G, ch), jnp.int32),
            pltpu.VMEM((G, ch), jnp.int32),
            pltpu.VMEM((n_acc,), jnp.float32),
            pltpu.VMEM((n_acc,), jnp.float32),
            pltpu.VMEM((L,), jnp.float32),
            pltpu.VMEM((L,), jnp.float32),
            pltpu.VMEM((2, ch, F), jnp.float32),
            pltpu.VMEM((ch,), jnp.float32),
            pltpu.VMEM_SHARED((n_rows, F), jnp.float32),
            pltpu.VMEM_SHARED((n_rows,), jnp.float32),
            pltpu.SemaphoreType.DMA,
            pltpu.SemaphoreType.DMA,
        ],
    )
    def gat(h_hbm, s_hbm, t_hbm, smax_hbm, tmax_hbm, eidx_hbm,
            num_hbm, den_hbm, sibuf, dibuf, sv, tv, smv, tmv, rows, ee,
            accn, accd, gs0, gs1):
        core = lax.axis_index("c")
        sub = lax.axis_index("s")
        wid = sub * NC + core

        z16 = jnp.zeros((L,), jnp.float32)

        def zrow(i, _):
            for j in range(F // L):
                rows[0, i, pl.ds(j * L, L)] = z16
            return 0

        lax.fori_loop(0, ch, zrow, 0)
        for j in range(ch // L):
            ee[pl.ds(j * L, L)] = z16

        def zblk(z, _):
            bz = sub + z * NS
            off = jnp.minimum(bz * ch, n_acc - ch)
            pltpu.sync_copy(rows.at[0], accn.at[pl.ds(off, ch), :])
            pltpu.sync_copy(ee, accd.at[pl.ds(off, ch)])
            return 0

        nz = (nzb - sub + NS - 1) // NS
        lax.fori_loop(0, nz, zblk, 0)

        if n_rows > n_acc:

            @pl.when(sub == 0)
            def _():
                pltpu.sync_copy(
                    rows.at[0, pl.ds(0, n_rows - n_acc), :],
                    accn.at[pl.ds(n_acc, n_rows - n_acc), :],
                )
                pltpu.sync_copy(
                    ee.at[pl.ds(0, n_rows - n_acc)],
                    accd.at[pl.ds(n_acc, n_rows - n_acc)],
                )

        pltpu.sync_copy(s_hbm, sv)
        pltpu.sync_copy(t_hbm, tv)
        pltpu.sync_copy(smax_hbm, smv)
        pltpu.sync_copy(tmax_hbm, tmv)

        plsc.subcore_barrier()

        cbound = jnp.maximum(smv[...] + tmv[...], 0.0)

        def gather(kk, b, sem):
            return pltpu.make_async_copy(
                h_hbm.at[sibuf.at[kk]], rows.at[b], sem
            )

        def process(kk, b):
            for j in range(ch // L):
                si = sibuf[kk, pl.ds(j * L, L)]
                di = dibuf[kk, pl.ds(j * L, L)]
                svv = plsc.load_gather(sv, [si])
                tvv = plsc.load_gather(tv, [di])
                e = svv + tvv
                e = jnp.maximum(e, 0.2 * e)
                ee[pl.ds(j * L, L)] = jnp.exp(e - cbound)

            def scale(i, _):
                w = plsc.load_gather(ee, [jnp.full((L,), 0, jnp.int32) + i])
                for j in range(F // L):
                    rows[b, i, pl.ds(j * L, L)] = rows[b, i, pl.ds(j * L, L)] * w
                return 0

            lax.fori_loop(0, ch, scale, 0)
            pltpu.sync_copy(rows.at[b], accn.at[dibuf.at[kk]], add=True)
            pltpu.sync_copy(ee, accd.at[dibuf.at[kk]], add=True)

        def grp(g, _):
            pltpu.sync_copy(eidx_hbm.at[0, wid, pl.ds(g * G, G)], sibuf)
            pltpu.sync_copy(eidx_hbm.at[1, wid, pl.ds(g * G, G)], dibuf)
            gather(0, 0, gs0).start()

            def pair(p, _):
                k0 = 2 * p
                k1 = k0 + 1
                gather(k1, 1, gs1).start()
                gather(k0, 0, gs0).wait()
                process(k0, 0)

                @pl.when(k1 + 1 < G)
                def _():
                    gather(k1 + 1, 0, gs0).start()

                gather(k1, 1, gs1).wait()
                process(k1, 1)
                return 0

            lax.fori_loop(0, G // 2, pair, 0)
            return 0

        lax.fori_loop(0, nchunks_per_tile // G, grp, 0)

        plsc.subcore_barrier()

        def oblk(z, _):
            bz = sub + z * NS
            off = jnp.minimum(bz * ZR, n_acc - ZR)
            pltpu.sync_copy(
                accn.at[pl.ds(off, ZR), :], num_hbm.at[core, pl.ds(off, ZR), :]
            )
            return 0

        no = (nob - sub + NS - 1) // NS
        lax.fori_loop(0, no, oblk, 0)

        def oblkd(z, _):
            bz = sub + z * NS
            off = jnp.minimum(bz * ch, n_acc - ch)
            pltpu.sync_copy(accd.at[pl.ds(off, ch)], ee)
            pltpu.sync_copy(ee, den_hbm.at[pl.ds(core * n_acc + off, ch)])
            return 0

        lax.fori_loop(0, nz, oblkd, 0)

    return gat


def _gin_mlp(x, agg, wq1, bq1, wq2, bq2, wd1, bd1, wd2, bd2, nq, outer_relu):
    n = x.shape[0]
    B = nq
    grid = _ceil_div(n, B)

    def body(x_ref, a_ref, wq1_ref, bq1_ref, wq2_ref, bq2_ref,
             wd1_ref, bd1_ref, wd2_ref, bd2_ref, o_ref):
        i = pl.program_id(0)
        isq = i == 0
        w1 = jnp.where(isq, wq1_ref[...], wd1_ref[...])
        b1 = jnp.where(isq, bq1_ref[...], bd1_ref[...])
        w2 = jnp.where(isq, wq2_ref[...], wd2_ref[...])
        b2 = jnp.where(isq, bq2_ref[...], bd2_ref[...])
        a = a_ref[...]
        xa = x_ref[...] + a[0] + a[1]
        h = jnp.maximum(
            jnp.dot(xa, w1, preferred_element_type=jnp.float32) + b1[None, :],
            0.0,
        )
        h = jnp.dot(h, w2, preferred_element_type=jnp.float32) + b2[None, :]
        if outer_relu:
            h = jnp.maximum(h, 0.0)
        o_ref[...] = h

    wspec = pl.BlockSpec((F, F), lambda i: (0, 0))
    bspec = pl.BlockSpec((F,), lambda i: (0,))
    return pl.pallas_call(
        body,
        grid=(grid,),
        in_specs=[
            pl.BlockSpec((B, F), lambda i: (i, 0)),
            pl.BlockSpec((NC, B, F), lambda i: (0, i, 0)),
            wspec, bspec, wspec, bspec, wspec, bspec, wspec, bspec,
        ],
        out_specs=pl.BlockSpec((B, F), lambda i: (i, 0)),
        out_shape=jax.ShapeDtypeStruct((n, F), jnp.float32),
    )(x, agg, wq1, bq1, wq2, bq2, wd1, bd1, wd2, bd2)


def _gat_pre(x, w, a_src, a_dst):
    n = x.shape[0]
    B = 1024
    grid = _ceil_div(n, B)
    neg = -3.0e38

    def body(x_ref, w_ref, as_ref, ad_ref, h_ref, s_ref, t_ref, sm_ref, tm_ref):
        i = pl.program_id(0)
        h = jnp.dot(x_ref[...], w_ref[...], preferred_element_type=jnp.float32)
        h_ref[...] = h
        s = jnp.dot(h, as_ref[...][:, None], preferred_element_type=jnp.float32)
        t = jnp.dot(h, ad_ref[...][:, None], preferred_element_type=jnp.float32)
        s_ref[...] = s
        t_ref[...] = t
        rows = i * B + lax.broadcasted_iota(jnp.int32, (B, 1), 0)
        valid = rows < n
        sm = jnp.max(jnp.where(valid, s, neg))
        tm = jnp.max(jnp.where(valid, t, neg))

        @pl.when(i == 0)
        def _():
            sm_ref[...] = jnp.full((L,), neg, jnp.float32)
            tm_ref[...] = jnp.full((L,), neg, jnp.float32)

        sm_ref[...] = jnp.maximum(sm_ref[...], sm)
        tm_ref[...] = jnp.maximum(tm_ref[...], tm)

    return pl.pallas_call(
        body,
        grid=(grid,),
        in_specs=[
            pl.BlockSpec((B, F), lambda i: (i, 0)),
            pl.BlockSpec((F, F), lambda i: (0, 0)),
            pl.BlockSpec((F,), lambda i: (0,)),
            pl.BlockSpec((F,), lambda i: (0,)),
        ],
        out_specs=[
            pl.BlockSpec((B, F), lambda i: (i, 0)),
            pl.BlockSpec((B, 1), lambda i: (i, 0)),
            pl.BlockSpec((B, 1), lambda i: (i, 0)),
            pl.BlockSpec((L,), lambda i: (0,)),
            pl.BlockSpec((L,), lambda i: (0,)),
        ],
        out_shape=[
            jax.ShapeDtypeStruct((n, F), jnp.float32),
            jax.ShapeDtypeStruct((n, 1), jnp.float32),
            jax.ShapeDtypeStruct((n, 1), jnp.float32),
            jax.ShapeDtypeStruct((L,), jnp.float32),
            jax.ShapeDtypeStruct((L,), jnp.float32),
        ],
    )(x, w, a_src, a_dst)


def _finalize(ginx, nump, denp, b, nq, nv, row0, nrows_out):
    B = nq
    grid = _ceil_div(nrows_out, B)
    ob = row0 // B

    def body(g_ref, n_ref, d_ref, b_ref, o_ref, sg_ref, sa_ref):
        i = pl.program_id(0)
        gx = g_ref[...]
        nsum = n_ref[...][0] + n_ref[...][1]
        den = d_ref[...][0] + d_ref[...][1] + 1e-16
        att = nsum / den[:, None] + b_ref[...][None, :]
        o_ref[:, 0:F] = gx
        o_ref[:, F : 2 * F] = att
        rows = row0 + i * B + lax.broadcasted_iota(jnp.int32, (B, 1), 0)
        valid = rows < nv

        @pl.when(i == 0)
        def _():
            sg_ref[...] = jnp.zeros((1, F), jnp.float32)
            sa_ref[...] = jnp.zeros((1, F), jnp.float32)

        sg_ref[...] += jnp.where(valid, gx, 0.0).sum(axis=0, keepdims=True)
        sa_ref[...] += jnp.where(valid, att, 0.0).sum(axis=0, keepdims=True)

    return pl.pallas_call(
        body,
        grid=(grid,),
        in_specs=[
            pl.BlockSpec((B, F), lambda i: (i + ob, 0)),
            pl.BlockSpec((NC, B, F), lambda i: (0, i + ob, 0)),
            pl.BlockSpec((NC, B), lambda i: (0, i + ob)),
            pl.BlockSpec((F,), lambda i: (0,)),
        ],
        out_specs=[
            pl.BlockSpec((B, 2 * F), lambda i: (i, 0)),
            pl.BlockSpec((1, F), lambda i: (0, 0)),
            pl.BlockSpec((1, F), lambda i: (0, 0)),
        ],
        out_shape=[
            jax.ShapeDtypeStruct((nrows_out, 2 * F), jnp.float32),
            jax.ShapeDtypeStruct((1, F), jnp.float32),
            jax.ShapeDtypeStruct((1, F), jnp.float32),
        ],
    )(ginx, nump, denp, b)


def _head(qa, qb, da, db, w1, b1, w2, b2, w3, b3, w4, b4):
    def body(qa_ref, qb_ref, da_ref, db_ref, w1_ref, b1_ref, w2_ref, b2_ref,
             w3_ref, b3_ref, w4_ref, b4_ref, o_ref):
        w1v = w1_ref[...]
        h = (
            jnp.dot(qa_ref[...], w1v[0:128], preferred_element_type=jnp.float32)
            + jnp.dot(qb_ref[...], w1v[128:256], preferred_element_type=jnp.float32)
            + jnp.dot(da_ref[...], w1v[256:384], preferred_element_type=jnp.float32)
            + jnp.dot(db_ref[...], w1v[384:512], preferred_element_type=jnp.float32)
            + b1_ref[...][None, :]
        )
        h = jnp.dot(h, w2_ref[...], preferred_element_type=jnp.float32) + b2_ref[...][None, :]
        h = jnp.maximum(h, 0.0)
        h = jnp.dot(h, w3_ref[...], preferred_element_type=jnp.float32) + b3_ref[...][None, :]
        h = jnp.maximum(h, 0.0)
        h = jnp.dot(h, w4_ref[...], preferred_element_type=jnp.float32) + b4_ref[...][None, :]
        o_ref[...] = jnp.maximum(h, 0.0)

    return pl.pallas_call(
        body,
        out_shape=jax.ShapeDtypeStruct((1, 1), jnp.float32),
    )(qa, qb, da, db, w1, b1, w2, b2, w3, b3, w4, b4)


def _prep_edges(src, dst, n_acc, ch):
    e = src.shape[0]
    unit = 2 * ch * NW
    epad = _ceil_div(e, unit) * unit
    npad = epad - e
    if npad:
        fill_src = (jnp.arange(npad, dtype=jnp.int32) % 64)
        fill_dst = n_acc + (jnp.arange(npad, dtype=jnp.int32) % 8)
        src = jnp.concatenate([src, fill_src])
        dst = jnp.concatenate([dst, fill_dst])
    per = epad // (NW * ch)
    src3 = src.reshape(NW, per, ch)
    dst3 = dst.reshape(NW, per, ch)
    eidx = jnp.stack([src3, dst3], axis=0)
    return eidx, per, npad


def kernel(query_in_feat, data_in_feat, query_edge_list, data_edge_list,
           query2data_edge_list, qg_W1, qg_b1, qg_W2, qg_b2, qg_W3, qg_b3,
           qg_W4, qg_b4, dg_W1, dg_b1, dg_W2, dg_b2, dg_W3, dg_b3, dg_W4,
           dg_b4, gat_W, gat_a_src, gat_a_dst, gat_b, L1_W, L1_b, L2_W, L2_b,
           L3_W, L3_b, L4_W, L4_b):
    nq = query_in_feat.shape[0]
    nd = data_in_feat.shape[0]
    ntot = nq + nd

    qe = query_edge_list.astype(jnp.int32)
    de = data_edge_list.astype(jnp.int32)
    xe = query2data_edge_list.astype(jnp.int32)

    csrc = jnp.concatenate([qe[0], de[0] + nq])
    cdst = jnp.concatenate([qe[1], de[1] + nq])
    cidx, cper, cpad = _prep_edges(csrc, cdst, ntot, CH_GIN)
    xidx, xper, xpad = _prep_edges(xe[0], xe[1], ntot, CH_GAT)

    segsum = _make_segsum(ntot + (8 if cpad else 0), ntot, cper, CH_GIN)
    gat_edges = _make_gat_edges(ntot + (8 if xpad else 0), ntot, xper, CH_GAT)

    x0 = jnp.concatenate([query_in_feat, data_in_feat], axis=0)

    agg1 = segsum(x0, cidx)
    h1 = _gin_mlp(x0, agg1, qg_W1, qg_b1, qg_W2, qg_b2,
                  dg_W1, dg_b1, dg_W2, dg_b2, nq, outer_relu=True)

    agg2 = segsum(h1, cidx)
    ginx = _gin_mlp(h1, agg2, qg_W3, qg_b3, qg_W4, qg_b4,
                    dg_W3, dg_b3, dg_W4, dg_b4, nq, outer_relu=False)

    hg, s, t, smax, tmax = _gat_pre(x0, gat_W, gat_a_src, gat_a_dst)
    nump, denp = gat_edges(hg, s[:, 0], t[:, 0], smax, tmax, xidx)
    denp2 = denp.reshape(NC, ntot)

    out_q, qsA, qsB = _finalize(ginx, nump, denp2, gat_b, nq, ntot, 0, nq)
    out_d, dsA, dsB = _finalize(ginx, nump, denp2, gat_b, nq, ntot, nq, nd)

    pred = _head(qsA, qsB, dsA, dsB, L1_W, L1_b, L2_W, L2_b, L3_W, L3_b,
                 L4_W, L4_b)

    return (pred, out_q, out_d)


# speedup vs baseline: 1.0120x; 1.0120x over previous
import functools

import jax
import jax.numpy as jnp
from jax import lax
from jax.experimental import pallas as pl
from jax.experimental.pallas import tpu as pltpu
from jax.experimental.pallas import tpu_sc as plsc

NC = 2
NS = 16
NW = NC * NS
L = 16

CH_GIN = 128
CH_GAT = 64
ZR = 128
F = 128


def _ceil_div(a, b):
    return (a + b - 1) // b


G = 16


def _make_segsum(n_rows, n_acc, nchunks_per_tile, ch):
    mesh = plsc.VectorSubcoreMesh(core_axis_name="c", subcore_axis_name="s")
    nzb = _ceil_div(n_acc, ch)
    nob = _ceil_div(n_acc, ZR)
    assert nchunks_per_tile % G == 0

    @functools.partial(
        pl.kernel,
        out_type=jax.ShapeDtypeStruct((NC, n_acc, F), jnp.float32),
        mesh=mesh,
        compiler_params=pltpu.CompilerParams(needs_layout_passes=False),
        scratch_types=[
            pltpu.VMEM((G, ch), jnp.int32),
            pltpu.VMEM((G, ch), jnp.int32),
            pltpu.VMEM((2, ch, F), jnp.float32),
            pltpu.VMEM_SHARED((n_rows, F), jnp.float32),
            pltpu.SemaphoreType.DMA,
            pltpu.SemaphoreType.DMA,
        ],
    )
    def segsum(x_hbm, eidx_hbm, out_hbm, sibuf, dibuf, rows, acc, gs0, gs1):
        core = lax.axis_index("c")
        sub = lax.axis_index("s")
        wid = sub * NC + core

        z16 = jnp.zeros((L,), jnp.float32)

        def zrow(i, _):
            for j in range(F // L):
                rows[0, i, pl.ds(j * L, L)] = z16
            return 0

        lax.fori_loop(0, ch, zrow, 0)

        def zblk(z, _):
            bz = sub + z * NS
            off = jnp.minimum(bz * ch, n_acc - ch)
            pltpu.sync_copy(rows.at[0], acc.at[pl.ds(off, ch), :])
            return 0

        nz = (nzb - sub + NS - 1) // NS
        lax.fori_loop(0, nz, zblk, 0)
        if n_rows > n_acc:

            @pl.when(sub == 0)
            def _():
                pltpu.sync_copy(
                    rows.at[0, pl.ds(0, n_rows - n_acc), :],
                    acc.at[pl.ds(n_acc, n_rows - n_acc), :],
                )

        plsc.subcore_barrier()

        def gather(kk, b, sem):
            return pltpu.make_async_copy(
                x_hbm.at[sibuf.at[kk]], rows.at[b], sem
            )

        def scat(kk, b):
            pltpu.sync_copy(rows.at[b], acc.at[dibuf.at[kk]], add=True)

        def grp(g, _):
            pltpu.sync_copy(eidx_hbm.at[0, wid, pl.ds(g * G, G)], sibuf)
            pltpu.sync_copy(eidx_hbm.at[1, wid, pl.ds(g * G, G)], dibuf)
            gather(0, 0, gs0).start()

            def pair(p, _):
                k0 = 2 * p
                k1 = k0 + 1
                gather(k1, 1, gs1).start()
                gather(k0, 0, gs0).wait()
                scat(k0, 0)

                @pl.when(k1 + 1 < G)
                def _():
                    gather(k1 + 1, 0, gs0).start()

                gather(k1, 1, gs1).wait()
                scat(k1, 1)
                return 0

            lax.fori_loop(0, G // 2, pair, 0)
            return 0

        lax.fori_loop(0, nchunks_per_tile // G, grp, 0)

        plsc.subcore_barrier()

        def oblk(z, _):
            bz = sub + z * NS
            off = jnp.minimum(bz * ZR, n_acc - ZR)
            pltpu.sync_copy(
                acc.at[pl.ds(off, ZR), :], out_hbm.at[core, pl.ds(off, ZR), :]
            )
            return 0

        no = (nob - sub + NS - 1) // NS
        lax.fori_loop(0, no, oblk, 0)

    return segsum


def _make_gat_edges(n_rows, n_acc, nchunks_per_tile, ch):
    mesh = plsc.VectorSubcoreMesh(core_axis_name="c", subcore_axis_name="s")
    nzb = _ceil_div(n_acc, ch)
    nob = _ceil_div(n_acc, ZR)
    assert nchunks_per_tile % G == 0

    @functools.partial(
        pl.kernel,
        out_type=(
            jax.ShapeDtypeStruct((NC, n_acc, F), jnp.float32),
            jax.ShapeDtypeStruct((NC * n_acc,), jnp.float32),
        ),
        mesh=mesh,
        compiler_params=pltpu.CompilerParams(needs_layout_passes=False),
        scratch_types=[
            pltpu.VMEM((G, ch), jnp.int32),
            pltpu.VMEM((G, ch), jnp.int32),
            pltpu.VMEM((n_acc,), jnp.float32),
            pltpu.VMEM((n_acc,), jnp.float32),
            pltpu.VMEM((L,), jnp.float32),
            pltpu.VMEM((L,), jnp.float32),
            pltpu.VMEM((2, ch, F), jnp.float32),
            pltpu.VMEM((ch,), jnp.float32),
            pltpu.VMEM_SHARED((n_rows, F), jnp.float32),
            pltpu.VMEM_SHARED((n_rows,), jnp.float32),
            pltpu.SemaphoreType.DMA,
            pltpu.SemaphoreType.DMA,
        ],
    )
    def gat(h_hbm, s_hbm, t_hbm, smax_hbm, tmax_hbm, eidx_hbm,
            num_hbm, den_hbm, sibuf, dibuf, sv, tv, smv, tmv, rows, ee,
            accn, accd, gs0, gs1):
        core = lax.axis_index("c")
        sub = lax.axis_index("s")
        wid = sub * NC + core

        z16 = jnp.zeros((L,), jnp.float32)

        def zrow(i, _):
            for j in range(F // L):
                rows[0, i, pl.ds(j * L, L)] = z16
            return 0

        lax.fori_loop(0, ch, zrow, 0)
        for j in range(ch // L):
            ee[pl.ds(j * L, L)] = z16

        def zblk(z, _):
            bz = sub + z * NS
            off = jnp.minimum(bz * ch, n_acc - ch)
            pltpu.sync_copy(rows.at[0], accn.at[pl.ds(off, ch), :])
            pltpu.sync_copy(ee, accd.at[pl.ds(off, ch)])
            return 0

        nz = (nzb - sub + NS - 1) // NS
        lax.fori_loop(0, nz, zblk, 0)

        if n_rows > n_acc:

            @pl.when(sub == 0)
            def _():
                pltpu.sync_copy(
                    rows.at[0, pl.ds(0, n_rows - n_acc), :],
                    accn.at[pl.ds(n_acc, n_rows - n_acc), :],
                )
                pltpu.sync_copy(
                    ee.at[pl.ds(0, n_rows - n_acc)],
                    accd.at[pl.ds(n_acc, n_rows - n_acc)],
                )

        pltpu.sync_copy(s_hbm, sv)
        pltpu.sync_copy(t_hbm, tv)
        pltpu.sync_copy(smax_hbm, smv)
        pltpu.sync_copy(tmax_hbm, tmv)

        plsc.subcore_barrier()

        cbound = jnp.maximum(smv[...] + tmv[...], 0.0)

        def gather(kk, b, sem):
            return pltpu.make_async_copy(
                h_hbm.at[sibuf.at[kk]], rows.at[b], sem
            )

        def process(kk, b):
            for j in range(ch // L):
                si = sibuf[kk, pl.ds(j * L, L)]
                di = dibuf[kk, pl.ds(j * L, L)]
                svv = plsc.load_gather(sv, [si])
                tvv = plsc.load_gather(tv, [di])
                e = svv + tvv
                e = jnp.maximum(e, 0.2 * e)
                ee[pl.ds(j * L, L)] = jnp.exp(e - cbound)

            def scale(i, _):
                w = plsc.load_gather(ee, [jnp.full((L,), 0, jnp.int32) + i])
                for j in range(F // L):
                    rows[b, i, pl.ds(j * L, L)] = rows[b, i, pl.ds(j * L, L)] * w
                return 0

            lax.fori_loop(0, ch, scale, 0)
            pltpu.sync_copy(rows.at[b], accn.at[dibuf.at[kk]], add=True)
            pltpu.sync_copy(ee, accd.at[dibuf.at[kk]], add=True)

        def grp(g, _):
            pltpu.sync_copy(eidx_hbm.at[0, wid, pl.ds(g * G, G)], sibuf)
            pltpu.sync_copy(eidx_hbm.at[1, wid, pl.ds(g * G, G)], dibuf)
            gather(0, 0, gs0).start()

            def pair(p, _):
                k0 = 2 * p
                k1 = k0 + 1
                gather(k1, 1, gs1).start()
                gather(k0, 0, gs0).wait()
                process(k0, 0)

                @pl.when(k1 + 1 < G)
                def _():
                    gather(k1 + 1, 0, gs0).start()

                gather(k1, 1, gs1).wait()
                process(k1, 1)
                return 0

            lax.fori_loop(0, G // 2, pair, 0)
            return 0

        lax.fori_loop(0, nchunks_per_tile // G, grp, 0)

        plsc.subcore_barrier()

        def oblk(z, _):
            bz = sub + z * NS
            off = jnp.minimum(bz * ZR, n_acc - ZR)
            pltpu.sync_copy(
                accn.at[pl.ds(off, ZR), :], num_hbm.at[core, pl.ds(off, ZR), :]
            )
            return 0

        no = (nob - sub + NS - 1) // NS
        lax.fori_loop(0, no, oblk, 0)

        def oblkd(z, _):
            bz = sub + z * NS
            off = jnp.minimum(bz * ch, n_acc - ch)
            pltpu.sync_copy(accd.at[pl.ds(off, ch)], ee)
            pltpu.sync_copy(ee, den_hbm.at[pl.ds(core * n_acc + off, ch)])
            return 0

        lax.fori_loop(0, nz, oblkd, 0)

    return gat


def _gin_mlp(x, agg, wq1, bq1, wq2, bq2, wd1, bd1, wd2, bd2, nq, outer_relu):
    n = x.shape[0]
    B = nq
    grid = _ceil_div(n, B)

    def body(x_ref, a_ref, wq1_ref, bq1_ref, wq2_ref, bq2_ref,
             wd1_ref, bd1_ref, wd2_ref, bd2_ref, o_ref):
        i = pl.program_id(0)
        isq = i == 0
        w1 = jnp.where(isq, wq1_ref[...], wd1_ref[...])
        b1 = jnp.where(isq, bq1_ref[...], bd1_ref[...])
        w2 = jnp.where(isq, wq2_ref[...], wd2_ref[...])
        b2 = jnp.where(isq, bq2_ref[...], bd2_ref[...])
        a = a_ref[...]
        xa = x_ref[...] + a[0] + a[1]
        h = jnp.maximum(
            jnp.dot(xa, w1, preferred_element_type=jnp.float32) + b1[None, :],
            0.0,
        )
        h = jnp.dot(h, w2, preferred_element_type=jnp.float32) + b2[None, :]
        if outer_relu:
            h = jnp.maximum(h, 0.0)
        o_ref[...] = h

    wspec = pl.BlockSpec((F, F), lambda i: (0, 0))
    bspec = pl.BlockSpec((F,), lambda i: (0,))
    return pl.pallas_call(
        body,
        grid=(grid,),
        in_specs=[
            pl.BlockSpec((B, F), lambda i: (i, 0)),
            pl.BlockSpec((NC, B, F), lambda i: (0, i, 0)),
            wspec, bspec, wspec, bspec, wspec, bspec, wspec, bspec,
        ],
        out_specs=pl.BlockSpec((B, F), lambda i: (i, 0)),
        out_shape=jax.ShapeDtypeStruct((n, F), jnp.float32),
    )(x, agg, wq1, bq1, wq2, bq2, wd1, bd1, wd2, bd2)


def _gat_pre(x, w, a_src, a_dst):
    n = x.shape[0]
    B = 1024
    grid = _ceil_div(n, B)
    neg = -3.0e38

    def body(x_ref, w_ref, as_ref, ad_ref, h_ref, s_ref, t_ref, sm_ref, tm_ref):
        i = pl.program_id(0)
        h = jnp.dot(x_ref[...], w_ref[...], preferred_element_type=jnp.float32)
        h_ref[...] = h
        s = jnp.dot(h, as_ref[...][:, None], preferred_element_type=jnp.float32)
        t = jnp.dot(h, ad_ref[...][:, None], preferred_element_type=jnp.float32)
        s_ref[...] = s
        t_ref[...] = t
        rows = i * B + lax.broadcasted_iota(jnp.int32, (B, 1), 0)
        valid = rows < n
        sm = jnp.max(jnp.where(valid, s, neg))
        tm = jnp.max(jnp.where(valid, t, neg))

        @pl.when(i == 0)
        def _():
            sm_ref[...] = jnp.full((L,), neg, jnp.float32)
            tm_ref[...] = jnp.full((L,), neg, jnp.float32)

        sm_ref[...] = jnp.maximum(sm_ref[...], sm)
        tm_ref[...] = jnp.maximum(tm_ref[...], tm)

    return pl.pallas_call(
        body,
        grid=(grid,),
        in_specs=[
            pl.BlockSpec((B, F), lambda i: (i, 0)),
            pl.BlockSpec((F, F), lambda i: (0, 0)),
            pl.BlockSpec((F,), lambda i: (0,)),
            pl.BlockSpec((F,), lambda i: (0,)),
        ],
        out_specs=[
            pl.BlockSpec((B, F), lambda i: (i, 0)),
            pl.BlockSpec((B, 1), lambda i: (i, 0)),
            pl.BlockSpec((B, 1), lambda i: (i, 0)),
            pl.BlockSpec((L,), lambda i: (0,)),
            pl.BlockSpec((L,), lambda i: (0,)),
        ],
        out_shape=[
            jax.ShapeDtypeStruct((n, F), jnp.float32),
            jax.ShapeDtypeStruct((n, 1), jnp.float32),
            jax.ShapeDtypeStruct((n, 1), jnp.float32),
            jax.ShapeDtypeStruct((L,), jnp.float32),
            jax.ShapeDtypeStruct((L,), jnp.float32),
        ],
    )(x, w, a_src, a_dst)


def _finalize(ginx, nump, denp, b, nq, nv, row0, nrows_out):
    B = nq
    grid = _ceil_div(nrows_out, B)
    ob = row0 // B

    def body(g_ref, n_ref, d_ref, b_ref, o_ref, sg_ref, sa_ref):
        i = pl.program_id(0)
        gx = g_ref[...]
        nsum = n_ref[...][0] + n_ref[...][1]
        den = d_ref[...][0] + d_ref[...][1] + 1e-16
        att = nsum / den[:, None] + b_ref[...][None, :]
        o_ref[:, 0:F] = gx
        o_ref[:, F : 2 * F] = att
        rows = row0 + i * B + lax.broadcasted_iota(jnp.int32, (B, 1), 0)
        valid = rows < nv

        @pl.when(i == 0)
        def _():
            sg_ref[...] = jnp.zeros((1, F), jnp.float32)
            sa_ref[...] = jnp.zeros((1, F), jnp.float32)

        sg_ref[...] += jnp.where(valid, gx, 0.0).sum(axis=0, keepdims=True)
        sa_ref[...] += jnp.where(valid, att, 0.0).sum(axis=0, keepdims=True)

    return pl.pallas_call(
        body,
        grid=(grid,),
        in_specs=[
            pl.BlockSpec((B, F), lambda i: (i + ob, 0)),
            pl.BlockSpec((NC, B, F), lambda i: (0, i + ob, 0)),
            pl.BlockSpec((NC, B), lambda i: (0, i + ob)),
            pl.BlockSpec((F,), lambda i: (0,)),
        ],
        out_specs=[
            pl.BlockSpec((B, 2 * F), lambda i: (i, 0)),
            pl.BlockSpec((1, F), lambda i: (0, 0)),
            pl.BlockSpec((1, F), lambda i: (0, 0)),
        ],
        out_shape=[
            jax.ShapeDtypeStruct((nrows_out, 2 * F), jnp.float32),
            jax.ShapeDtypeStruct((1, F), jnp.float32),
            jax.ShapeDtypeStruct((1, F), jnp.float32),
        ],
    )(ginx, nump, denp, b)


def _head(qa, qb, da, db, w1, b1, w2, b2, w3, b3, w4, b4):
    def body(qa_ref, qb_ref, da_ref, db_ref, w1_ref, b1_ref, w2_ref, b2_ref,
             w3_ref, b3_ref, w4_ref, b4_ref, o_ref):
        w1v = w1_ref[...]
        h = (
            jnp.dot(qa_ref[...], w1v[0:128], preferred_element_type=jnp.float32)
            + jnp.dot(qb_ref[...], w1v[128:256], preferred_element_type=jnp.float32)
            + jnp.dot(da_ref[...], w1v[256:384], preferred_element_type=jnp.float32)
            + jnp.dot(db_ref[...], w1v[384:512], preferred_element_type=jnp.float32)
            + b1_ref[...][None, :]
        )
        h = jnp.dot(h, w2_ref[...], preferred_element_type=jnp.float32) + b2_ref[...][None, :]
        h = jnp.maximum(h, 0.0)
        h = jnp.dot(h, w3_ref[...], preferred_element_type=jnp.float32) + b3_ref[...][None, :]
        h = jnp.maximum(h, 0.0)
        h = jnp.dot(h, w4_ref[...], preferred_element_type=jnp.float32) + b4_ref[...][None, :]
        o_ref[...] = jnp.maximum(h, 0.0)

    return pl.pallas_call(
        body,
        out_shape=jax.ShapeDtypeStruct((1, 1), jnp.float32),
    )(qa, qb, da, db, w1, b1, w2, b2, w3, b3, w4, b4)


def _prep_edges(edge2, n_acc, ch):
    e = edge2.shape[1]
    unit = 2 * ch * NW
    epad = _ceil_div(e, unit) * unit
    npad = epad - e
    if npad:
        ar = jnp.arange(npad, dtype=jnp.int32)
        pad2 = jnp.stack([ar % 64, n_acc + ar % 8], axis=0)
        edge2 = jnp.concatenate([edge2, pad2], axis=1)
    per = epad // (NW * ch)
    eidx = edge2.reshape(2, NW, per, ch)
    return eidx, per, npad


def kernel(query_in_feat, data_in_feat, query_edge_list, data_edge_list,
           query2data_edge_list, qg_W1, qg_b1, qg_W2, qg_b2, qg_W3, qg_b3,
           qg_W4, qg_b4, dg_W1, dg_b1, dg_W2, dg_b2, dg_W3, dg_b3, dg_W4,
           dg_b4, gat_W, gat_a_src, gat_a_dst, gat_b, L1_W, L1_b, L2_W, L2_b,
           L3_W, L3_b, L4_W, L4_b):
    nq = query_in_feat.shape[0]
    nd = data_in_feat.shape[0]
    ntot = nq + nd

    qe = query_edge_list.astype(jnp.int32)
    de = data_edge_list.astype(jnp.int32)
    xe = query2data_edge_list.astype(jnp.int32)

    cedge = jnp.concatenate([qe, de + nq], axis=1)
    cidx, cper, cpad = _prep_edges(cedge, ntot, CH_GIN)
    xidx, xper, xpad = _prep_edges(xe, ntot, CH_GAT)

    segsum = _make_segsum(ntot + (8 if cpad else 0), ntot, cper, CH_GIN)
    gat_edges = _make_gat_edges(ntot + (8 if xpad else 0), ntot, xper, CH_GAT)

    x0 = jnp.concatenate([query_in_feat, data_in_feat], axis=0)

    agg1 = segsum(x0, cidx)
    h1 = _gin_mlp(x0, agg1, qg_W1, qg_b1, qg_W2, qg_b2,
                  dg_W1, dg_b1, dg_W2, dg_b2, nq, outer_relu=True)

    agg2 = segsum(h1, cidx)
    ginx = _gin_mlp(h1, agg2, qg_W3, qg_b3, qg_W4, qg_b4,
                    dg_W3, dg_b3, dg_W4, dg_b4, nq, outer_relu=False)

    hg, s, t, smax, tmax = _gat_pre(x0, gat_W, gat_a_src, gat_a_dst)
    nump, denp = gat_edges(hg, s[:, 0], t[:, 0], smax, tmax, xidx)
    denp2 = denp.reshape(NC, ntot)

    out_q, qsA, qsB = _finalize(ginx, nump, denp2, gat_b, nq, ntot, 0, nq)
    out_d, dsA, dsB = _finalize(ginx, nump, denp2, gat_b, nq, ntot, nq, nd)

    pred = _head(qsA, qsB, dsA, dsB, L1_W, L1_b, L2_W, L2_b, L3_W, L3_b,
                 L4_W, L4_b)

    return (pred, out_q, out_d)


# speedup vs baseline: 1.0510x; 1.0386x over previous
import functools

import jax
import jax.numpy as jnp
from jax import lax
from jax.experimental import pallas as pl
from jax.experimental.pallas import tpu as pltpu
from jax.experimental.pallas import tpu_sc as plsc

NC = 2
NS = 16
NW = NC * NS
L = 16

CH_GIN = 128
CH_GAT = 64
ZR = 128
F = 128


def _ceil_div(a, b):
    return (a + b - 1) // b


G = 16


def _make_segsum(n_rows, n_acc, nchunks_per_tile, ch):
    mesh = plsc.VectorSubcoreMesh(core_axis_name="c", subcore_axis_name="s")
    nzb = _ceil_div(n_acc, ch)
    nob = _ceil_div(n_acc, ZR)
    assert nchunks_per_tile % G == 0

    @functools.partial(
        pl.kernel,
        out_type=jax.ShapeDtypeStruct((NC, n_acc, F), jnp.float32),
        mesh=mesh,
        compiler_params=pltpu.CompilerParams(needs_layout_passes=False),
        scratch_types=[
            pltpu.VMEM((G, ch), jnp.int32),
            pltpu.VMEM((G, ch), jnp.int32),
            pltpu.VMEM((2, ch, F), jnp.float32),
            pltpu.VMEM_SHARED((n_rows, F), jnp.float32),
            pltpu.SemaphoreType.DMA,
            pltpu.SemaphoreType.DMA,
        ],
    )
    def segsum(x_hbm, eidx_hbm, out_hbm, sibuf, dibuf, rows, acc, gs0, gs1):
        core = lax.axis_index("c")
        sub = lax.axis_index("s")
        wid = sub * NC + core

        z16 = jnp.zeros((L,), jnp.float32)

        def zrow(i, _):
            for j in range(F // L):
                rows[0, i, pl.ds(j * L, L)] = z16
            return 0

        lax.fori_loop(0, ch, zrow, 0)

        def zblk(z, _):
            bz = sub + z * NS
            off = jnp.minimum(bz * ch, n_acc - ch)
            pltpu.sync_copy(rows.at[0], acc.at[pl.ds(off, ch), :])
            return 0

        nz = (nzb - sub + NS - 1) // NS
        lax.fori_loop(0, nz, zblk, 0)
        if n_rows > n_acc:

            @pl.when(sub == 0)
            def _():
                pltpu.sync_copy(
                    rows.at[0, pl.ds(0, n_rows - n_acc), :],
                    acc.at[pl.ds(n_acc, n_rows - n_acc), :],
                )

        plsc.subcore_barrier()

        def gather(kk, b, sem):
            return pltpu.make_async_copy(
                x_hbm.at[sibuf.at[kk]], rows.at[b], sem
            )

        def scat(kk, b):
            pltpu.sync_copy(rows.at[b], acc.at[dibuf.at[kk]], add=True)

        def grp(g, _):
            pltpu.sync_copy(eidx_hbm.at[0, wid, pl.ds(g * G, G)], sibuf)
            pltpu.sync_copy(eidx_hbm.at[1, wid, pl.ds(g * G, G)], dibuf)
            gather(0, 0, gs0).start()

            def pair(p, _):
                k0 = 2 * p
                k1 = k0 + 1
                gather(k1, 1, gs1).start()
                gather(k0, 0, gs0).wait()
                scat(k0, 0)

                @pl.when(k1 + 1 < G)
                def _():
                    gather(k1 + 1, 0, gs0).start()

                gather(k1, 1, gs1).wait()
                scat(k1, 1)
                return 0

            lax.fori_loop(0, G // 2, pair, 0)
            return 0

        lax.fori_loop(0, nchunks_per_tile // G, grp, 0)

        plsc.subcore_barrier()

        def oblk(z, _):
            bz = sub + z * NS
            off = jnp.minimum(bz * ZR, n_acc - ZR)
            pltpu.sync_copy(
                acc.at[pl.ds(off, ZR), :], out_hbm.at[core, pl.ds(off, ZR), :]
            )
            return 0

        no = (nob - sub + NS - 1) // NS
        lax.fori_loop(0, no, oblk, 0)

    return segsum


def _make_gat_edges(n_rows, n_acc, nchunks_per_tile, ch):
    mesh = plsc.VectorSubcoreMesh(core_axis_name="c", subcore_axis_name="s")
    nzb = _ceil_div(n_acc, ch)
    nob = _ceil_div(n_acc, ZR)
    assert nchunks_per_tile % G == 0

    @functools.partial(
        pl.kernel,
        out_type=(
            jax.ShapeDtypeStruct((NC, n_acc, F), jnp.float32),
            jax.ShapeDtypeStruct((NC * n_acc,), jnp.float32),
        ),
        mesh=mesh,
        compiler_params=pltpu.CompilerParams(needs_layout_passes=False),
        scratch_types=[
            pltpu.VMEM((G, ch), jnp.int32),
            pltpu.VMEM((G, ch), jnp.int32),
            pltpu.VMEM((n_acc,), jnp.float32),
            pltpu.VMEM((n_acc,), jnp.float32),
            pltpu.VMEM((L,), jnp.float32),
            pltpu.VMEM((L,), jnp.float32),
            pltpu.VMEM((2, ch, F), jnp.float32),
            pltpu.VMEM((ch,), jnp.float32),
            pltpu.VMEM_SHARED((n_rows, F), jnp.float32),
            pltpu.VMEM_SHARED((n_rows,), jnp.float32),
            pltpu.SemaphoreType.DMA,
            pltpu.SemaphoreType.DMA,
        ],
    )
    def gat(h_hbm, s_hbm, t_hbm, smax_hbm, tmax_hbm, eidx_hbm,
            num_hbm, den_hbm, sibuf, dibuf, sv, tv, smv, tmv, rows, ee,
            accn, accd, gs0, gs1):
        core = lax.axis_index("c")
        sub = lax.axis_index("s")
        wid = sub * NC + core

        z16 = jnp.zeros((L,), jnp.float32)

        def zrow(i, _):
            for j in range(F // L):
                rows[0, i, pl.ds(j * L, L)] = z16
            return 0

        lax.fori_loop(0, ch, zrow, 0)
        for j in range(ch // L):
            ee[pl.ds(j * L, L)] = z16

        def zblk(z, _):
            bz = sub + z * NS
            off = jnp.minimum(bz * ch, n_acc - ch)
            pltpu.sync_copy(rows.at[0], accn.at[pl.ds(off, ch), :])
            pltpu.sync_copy(ee, accd.at[pl.ds(off, ch)])
            return 0

        nz = (nzb - sub + NS - 1) // NS
        lax.fori_loop(0, nz, zblk, 0)

        if n_rows > n_acc:

            @pl.when(sub == 0)
            def _():
                pltpu.sync_copy(
                    rows.at[0, pl.ds(0, n_rows - n_acc), :],
                    accn.at[pl.ds(n_acc, n_rows - n_acc), :],
                )
                pltpu.sync_copy(
                    ee.at[pl.ds(0, n_rows - n_acc)],
                    accd.at[pl.ds(n_acc, n_rows - n_acc)],
                )

        pltpu.sync_copy(s_hbm, sv)
        pltpu.sync_copy(t_hbm, tv)
        pltpu.sync_copy(smax_hbm, smv)
        pltpu.sync_copy(tmax_hbm, tmv)

        plsc.subcore_barrier()

        cbound = jnp.maximum(smv[...] + tmv[...], 0.0)

        def gather(kk, b, sem):
            return pltpu.make_async_copy(
                h_hbm.at[sibuf.at[kk]], rows.at[b], sem
            )

        def process(kk, b):
            for j in range(ch // L):
                si = sibuf[kk, pl.ds(j * L, L)]
                di = dibuf[kk, pl.ds(j * L, L)]
                svv = plsc.load_gather(sv, [si])
                tvv = plsc.load_gather(tv, [di])
                e = svv + tvv
                e = jnp.maximum(e, 0.2 * e)
                ee[pl.ds(j * L, L)] = jnp.exp(e - cbound)

            def scale(i, _):
                w = plsc.load_gather(ee, [jnp.full((L,), 0, jnp.int32) + i])
                for j in range(F // L):
                    rows[b, i, pl.ds(j * L, L)] = rows[b, i, pl.ds(j * L, L)] * w
                return 0

            lax.fori_loop(0, ch, scale, 0)
            pltpu.sync_copy(rows.at[b], accn.at[dibuf.at[kk]], add=True)
            pltpu.sync_copy(ee, accd.at[dibuf.at[kk]], add=True)

        def grp(g, _):
            pltpu.sync_copy(eidx_hbm.at[0, wid, pl.ds(g * G, G)], sibuf)
            pltpu.sync_copy(eidx_hbm.at[1, wid, pl.ds(g * G, G)], dibuf)
            gather(0, 0, gs0).start()

            def pair(p, _):
                k0 = 2 * p
                k1 = k0 + 1
                gather(k1, 1, gs1).start()
                gather(k0, 0, gs0).wait()
                process(k0, 0)

                @pl.when(k1 + 1 < G)
                def _():
                    gather(k1 + 1, 0, gs0).start()

                gather(k1, 1, gs1).wait()
                process(k1, 1)
                return 0

            lax.fori_loop(0, G // 2, pair, 0)
            return 0

        lax.fori_loop(0, nchunks_per_tile // G, grp, 0)

        plsc.subcore_barrier()

        def oblk(z, _):
            bz = sub + z * NS
            off = jnp.minimum(bz * ZR, n_acc - ZR)
            pltpu.sync_copy(
                accn.at[pl.ds(off, ZR), :], num_hbm.at[core, pl.ds(off, ZR), :]
            )
            return 0

        no = (nob - sub + NS - 1) // NS
        lax.fori_loop(0, no, oblk, 0)

        def oblkd(z, _):
            bz = sub + z * NS
            off = jnp.minimum(bz * ch, n_acc - ch)
            pltpu.sync_copy(accd.at[pl.ds(off, ch)], ee)
            pltpu.sync_copy(ee, den_hbm.at[pl.ds(core * n_acc + off, ch)])
            return 0

        lax.fori_loop(0, nz, oblkd, 0)

    return gat


def _gin_mlp(x, agg, wq1, bq1, wq2, bq2, wd1, bd1, wd2, bd2, nq, outer_relu):
    n = x.shape[0]
    B = nq
    grid = _ceil_div(n, B)

    def body(x_ref, a_ref, wq1_ref, bq1_ref, wq2_ref, bq2_ref,
             wd1_ref, bd1_ref, wd2_ref, bd2_ref, o_ref):
        i = pl.program_id(0)
        isq = i == 0
        w1 = jnp.where(isq, wq1_ref[...], wd1_ref[...])
        b1 = jnp.where(isq, bq1_ref[...], bd1_ref[...])
        w2 = jnp.where(isq, wq2_ref[...], wd2_ref[...])
        b2 = jnp.where(isq, bq2_ref[...], bd2_ref[...])
        a = a_ref[...]
        xa = x_ref[...] + a[0] + a[1]
        h = jnp.maximum(
            jnp.dot(xa, w1, preferred_element_type=jnp.float32) + b1[None, :],
            0.0,
        )
        h = jnp.dot(h, w2, preferred_element_type=jnp.float32) + b2[None, :]
        if outer_relu:
            h = jnp.maximum(h, 0.0)
        o_ref[...] = h

    wspec = pl.BlockSpec((F, F), lambda i: (0, 0))
    bspec = pl.BlockSpec((F,), lambda i: (0,))
    return pl.pallas_call(
        body,
        grid=(grid,),
        in_specs=[
            pl.BlockSpec((B, F), lambda i: (i, 0)),
            pl.BlockSpec((NC, B, F), lambda i: (0, i, 0)),
            wspec, bspec, wspec, bspec, wspec, bspec, wspec, bspec,
        ],
        out_specs=pl.BlockSpec((B, F), lambda i: (i, 0)),
        out_shape=jax.ShapeDtypeStruct((n, F), jnp.float32),
    )(x, agg, wq1, bq1, wq2, bq2, wd1, bd1, wd2, bd2)


def _gat_pre(x, w, a_src, a_dst):
    n = x.shape[0]
    B = 1024
    grid = _ceil_div(n, B)
    neg = -3.0e38

    def body(x_ref, w_ref, as_ref, ad_ref, h_ref, s_ref, t_ref, sm_ref, tm_ref):
        i = pl.program_id(0)
        h = jnp.dot(x_ref[...], w_ref[...], preferred_element_type=jnp.float32)
        h_ref[...] = h
        s = jnp.dot(h, as_ref[...][:, None], preferred_element_type=jnp.float32)
        t = jnp.dot(h, ad_ref[...][:, None], preferred_element_type=jnp.float32)
        s_ref[...] = s
        t_ref[...] = t
        rows = i * B + lax.broadcasted_iota(jnp.int32, (B, 1), 0)
        valid = rows < n
        sm = jnp.max(jnp.where(valid, s, neg))
        tm = jnp.max(jnp.where(valid, t, neg))

        @pl.when(i == 0)
        def _():
            sm_ref[...] = jnp.full((L,), neg, jnp.float32)
            tm_ref[...] = jnp.full((L,), neg, jnp.float32)

        sm_ref[...] = jnp.maximum(sm_ref[...], sm)
        tm_ref[...] = jnp.maximum(tm_ref[...], tm)

    return pl.pallas_call(
        body,
        grid=(grid,),
        in_specs=[
            pl.BlockSpec((B, F), lambda i: (i, 0)),
            pl.BlockSpec((F, F), lambda i: (0, 0)),
            pl.BlockSpec((F,), lambda i: (0,)),
            pl.BlockSpec((F,), lambda i: (0,)),
        ],
        out_specs=[
            pl.BlockSpec((B, F), lambda i: (i, 0)),
            pl.BlockSpec((B, 1), lambda i: (i, 0)),
            pl.BlockSpec((B, 1), lambda i: (i, 0)),
            pl.BlockSpec((L,), lambda i: (0,)),
            pl.BlockSpec((L,), lambda i: (0,)),
        ],
        out_shape=[
            jax.ShapeDtypeStruct((n, F), jnp.float32),
            jax.ShapeDtypeStruct((n, 1), jnp.float32),
            jax.ShapeDtypeStruct((n, 1), jnp.float32),
            jax.ShapeDtypeStruct((L,), jnp.float32),
            jax.ShapeDtypeStruct((L,), jnp.float32),
        ],
    )(x, w, a_src, a_dst)


def _att_fin(nump, denp, b, nq, ntot):
    B = 1024
    grid = _ceil_div(ntot, B)

    def body(n_ref, d_ref, b_ref, att_ref, qs_ref, ds_ref):
        i = pl.program_id(0)
        nsum = n_ref[...][0] + n_ref[...][1]
        den = d_ref[...][0] + d_ref[...][1] + 1e-16
        att = nsum / den[:, None] + b_ref[...][None, :]
        att_ref[...] = att
        rows = i * B + lax.broadcasted_iota(jnp.int32, (B, 1), 0)
        attv = jnp.where(rows < ntot, att, 0.0)
        qm = rows < nq

        @pl.when(i == 0)
        def _():
            qs_ref[...] = jnp.zeros((1, F), jnp.float32)
            ds_ref[...] = jnp.zeros((1, F), jnp.float32)

        qs_ref[...] += jnp.where(qm, attv, 0.0).sum(axis=0, keepdims=True)
        ds_ref[...] += jnp.where(qm, 0.0, attv).sum(axis=0, keepdims=True)

    return pl.pallas_call(
        body,
        grid=(grid,),
        in_specs=[
            pl.BlockSpec((NC, B, F), lambda i: (0, i, 0)),
            pl.BlockSpec((NC, B), lambda i: (0, i)),
            pl.BlockSpec((F,), lambda i: (0,)),
        ],
        out_specs=[
            pl.BlockSpec((B, F), lambda i: (i, 0)),
            pl.BlockSpec((1, F), lambda i: (0, 0)),
            pl.BlockSpec((1, F), lambda i: (0, 0)),
        ],
        out_shape=[
            jax.ShapeDtypeStruct((ntot, F), jnp.float32),
            jax.ShapeDtypeStruct((1, F), jnp.float32),
            jax.ShapeDtypeStruct((1, F), jnp.float32),
        ],
    )(nump, denp, b)


def _gin_mlp_out(x, agg, wq1, bq1, wq2, bq2, wd1, bd1, wd2, bd2, att, nq, nd):
    n = x.shape[0]
    B = nq
    grid = _ceil_div(n, B)

    def body(x_ref, a_ref, wq1_ref, bq1_ref, wq2_ref, bq2_ref,
             wd1_ref, bd1_ref, wd2_ref, bd2_ref, att_ref,
             oq_ref, od_ref, sq_ref, sd_ref):
        i = pl.program_id(0)
        isq = i == 0
        w1 = jnp.where(isq, wq1_ref[...], wd1_ref[...])
        b1 = jnp.where(isq, bq1_ref[...], bd1_ref[...])
        w2 = jnp.where(isq, wq2_ref[...], wd2_ref[...])
        b2 = jnp.where(isq, bq2_ref[...], bd2_ref[...])
        a = a_ref[...]
        xa = x_ref[...] + a[0] + a[1]
        h = jnp.maximum(
            jnp.dot(xa, w1, preferred_element_type=jnp.float32) + b1[None, :],
            0.0,
        )
        h = jnp.dot(h, w2, preferred_element_type=jnp.float32) + b2[None, :]
        att = att_ref[...]

        @pl.when(isq)
        def _():
            oq_ref[:, 0:F] = h
            oq_ref[:, F:2 * F] = att
            sq_ref[...] = h.sum(axis=0, keepdims=True)
            sd_ref[...] = jnp.zeros((1, F), jnp.float32)

        @pl.when(jnp.logical_not(isq))
        def _():
            od_ref[:, 0:F] = h
            od_ref[:, F:2 * F] = att
            rows = i * B + lax.broadcasted_iota(jnp.int32, (B, 1), 0)
            sd_ref[...] += jnp.where(rows < n, h, 0.0).sum(axis=0, keepdims=True)

    return pl.pallas_call(
        body,
        grid=(grid,),
        in_specs=[
            pl.BlockSpec((B, F), lambda i: (i, 0)),
            pl.BlockSpec((NC, B, F), lambda i: (0, i, 0)),
            pl.BlockSpec((F, F), lambda i: (0, 0)),
            pl.BlockSpec((F,), lambda i: (0,)),
            pl.BlockSpec((F, F), lambda i: (0, 0)),
            pl.BlockSpec((F,), lambda i: (0,)),
            pl.BlockSpec((F, F), lambda i: (0, 0)),
            pl.BlockSpec((F,), lambda i: (0,)),
            pl.BlockSpec((F, F), lambda i: (0, 0)),
            pl.BlockSpec((F,), lambda i: (0,)),
            pl.BlockSpec((B, F), lambda i: (i, 0)),
        ],
        out_specs=[
            pl.BlockSpec((B, 2 * F), lambda i: (0, 0)),
            pl.BlockSpec((B, 2 * F), lambda i: (jnp.maximum(i - 1, 0), 0)),
            pl.BlockSpec((1, F), lambda i: (0, 0)),
            pl.BlockSpec((1, F), lambda i: (0, 0)),
        ],
        out_shape=[
            jax.ShapeDtypeStruct((nq, 2 * F), jnp.float32),
            jax.ShapeDtypeStruct((nd, 2 * F), jnp.float32),
            jax.ShapeDtypeStruct((1, F), jnp.float32),
            jax.ShapeDtypeStruct((1, F), jnp.float32),
        ],
    )(x, agg, wq1, bq1, wq2, bq2, wd1, bd1, wd2, bd2, att)


def _head(qa, qb, da, db, w1, b1, w2, b2, w3, b3, w4, b4):
    def body(qa_ref, qb_ref, da_ref, db_ref, w1_ref, b1_ref, w2_ref, b2_ref,
             w3_ref, b3_ref, w4_ref, b4_ref, o_ref):
        w1v = w1_ref[...]
        h = (
            jnp.dot(qa_ref[...], w1v[0:128], preferred_element_type=jnp.float32)
            + jnp.dot(qb_ref[...], w1v[128:256], preferred_element_type=jnp.float32)
            + jnp.dot(da_ref[...], w1v[256:384], preferred_element_type=jnp.float32)
            + jnp.dot(db_ref[...], w1v[384:512], preferred_element_type=jnp.float32)
            + b1_ref[...][None, :]
        )
        h = jnp.dot(h, w2_ref[...], preferred_element_type=jnp.float32) + b2_ref[...][None, :]
        h = jnp.maximum(h, 0.0)
        h = jnp.dot(h, w3_ref[...], preferred_element_type=jnp.float32) + b3_ref[...][None, :]
        h = jnp.maximum(h, 0.0)
        h = jnp.dot(h, w4_ref[...], preferred_element_type=jnp.float32) + b4_ref[...][None, :]
        o_ref[...] = jnp.maximum(h, 0.0)

    return pl.pallas_call(
        body,
        out_shape=jax.ShapeDtypeStruct((1, 1), jnp.float32),
    )(qa, qb, da, db, w1, b1, w2, b2, w3, b3, w4, b4)


def _prep_edges(edge2, n_acc, ch):
    e = edge2.shape[1]
    unit = 2 * ch * NW
    epad = _ceil_div(e, unit) * unit
    npad = epad - e
    if npad:
        ar = jnp.arange(npad, dtype=jnp.int32)
        pad2 = jnp.stack([ar % 64, n_acc + ar % 8], axis=0)
        edge2 = jnp.concatenate([edge2, pad2], axis=1)
    per = epad // (NW * ch)
    eidx = edge2.reshape(2, NW, per, ch)
    return eidx, per, npad


def kernel(query_in_feat, data_in_feat, query_edge_list, data_edge_list,
           query2data_edge_list, qg_W1, qg_b1, qg_W2, qg_b2, qg_W3, qg_b3,
           qg_W4, qg_b4, dg_W1, dg_b1, dg_W2, dg_b2, dg_W3, dg_b3, dg_W4,
           dg_b4, gat_W, gat_a_src, gat_a_dst, gat_b, L1_W, L1_b, L2_W, L2_b,
           L3_W, L3_b, L4_W, L4_b):
    nq = query_in_feat.shape[0]
    nd = data_in_feat.shape[0]
    ntot = nq + nd

    qe = query_edge_list.astype(jnp.int32)
    de = data_edge_list.astype(jnp.int32)
    xe = query2data_edge_list.astype(jnp.int32)

    cedge = jnp.concatenate([qe, de + nq], axis=1)
    cidx, cper, cpad = _prep_edges(cedge, ntot, CH_GIN)
    xidx, xper, xpad = _prep_edges(xe, ntot, CH_GAT)

    segsum = _make_segsum(ntot + (8 if cpad else 0), ntot, cper, CH_GIN)
    gat_edges = _make_gat_edges(ntot + (8 if xpad else 0), ntot, xper, CH_GAT)

    x0 = jnp.concatenate([query_in_feat, data_in_feat], axis=0)

    agg1 = segsum(x0, cidx)
    h1 = _gin_mlp(x0, agg1, qg_W1, qg_b1, qg_W2, qg_b2,
                  dg_W1, dg_b1, dg_W2, dg_b2, nq, outer_relu=True)

    hg, s, t, smax, tmax = _gat_pre(x0, gat_W, gat_a_src, gat_a_dst)
    nump, denp = gat_edges(hg, s[:, 0], t[:, 0], smax, tmax, xidx)
    denp2 = denp.reshape(NC, ntot)
    att, qsB, dsB = _att_fin(nump, denp2, gat_b, nq, ntot)

    agg2 = segsum(h1, cidx)
    out_q, out_d, qsA, dsA = _gin_mlp_out(
        h1, agg2, qg_W3, qg_b3, qg_W4, qg_b4,
        dg_W3, dg_b3, dg_W4, dg_b4, att, nq, nd)

    pred = _head(qsA, qsB, dsA, dsB, L1_W, L1_b, L2_W, L2_b, L3_W, L3_b,
                 L4_W, L4_b)

    return (pred, out_q, out_d)


# speedup vs baseline: 1.0546x; 1.0034x over previous
import functools

import jax
import jax.numpy as jnp
from jax import lax
from jax.experimental import pallas as pl
from jax.experimental.pallas import tpu as pltpu
from jax.experimental.pallas import tpu_sc as plsc

NC = 2
NS = 16
NW = NC * NS
L = 16

CH_GIN = 128
CH_GAT = 64
ZR = 128
F = 128


def _ceil_div(a, b):
    return (a + b - 1) // b


G = 16


def _make_segsum(n_rows, n_acc, nchunks_per_tile, ch):
    mesh = plsc.VectorSubcoreMesh(core_axis_name="c", subcore_axis_name="s")
    nzb = _ceil_div(n_acc, ch)
    nob = _ceil_div(n_acc, ZR)
    assert nchunks_per_tile % G == 0

    @functools.partial(
        pl.kernel,
        out_type=jax.ShapeDtypeStruct((NC, n_acc, F), jnp.float32),
        mesh=mesh,
        compiler_params=pltpu.CompilerParams(needs_layout_passes=False),
        scratch_types=[
            pltpu.VMEM((G, ch), jnp.int32),
            pltpu.VMEM((G, ch), jnp.int32),
            pltpu.VMEM((2, ch, F), jnp.float32),
            pltpu.VMEM_SHARED((n_rows, F), jnp.float32),
            pltpu.SemaphoreType.DMA,
            pltpu.SemaphoreType.DMA,
        ],
    )
    def segsum(x_hbm, eidx_hbm, out_hbm, sibuf, dibuf, rows, acc, gs0, gs1):
        core = lax.axis_index("c")
        sub = lax.axis_index("s")
        wid = sub * NC + core

        z16 = jnp.zeros((L,), jnp.float32)

        def zrow(i, _):
            for j in range(F // L):
                rows[0, i, pl.ds(j * L, L)] = z16
            return 0

        lax.fori_loop(0, ch, zrow, 0)

        def zdesc(z):
            bz = sub + z * NS
            off = jnp.minimum(bz * ch, n_acc - ch)
            return pltpu.make_async_copy(rows.at[0], acc.at[pl.ds(off, ch), :], gs1)

        nz = (nzb - sub + NS - 1) // NS
        lax.fori_loop(0, nz, lambda z, _: (zdesc(z).start(), 0)[1], 0)
        lax.fori_loop(0, nz, lambda z, _: (zdesc(z).wait(), 0)[1], 0)
        if n_rows > n_acc:

            @pl.when(sub == 0)
            def _():
                pltpu.sync_copy(
                    rows.at[0, pl.ds(0, n_rows - n_acc), :],
                    acc.at[pl.ds(n_acc, n_rows - n_acc), :],
                )

        plsc.subcore_barrier()

        def gather(kk, b, sem):
            return pltpu.make_async_copy(
                x_hbm.at[sibuf.at[kk]], rows.at[b], sem
            )

        def scat(kk, b):
            pltpu.sync_copy(rows.at[b], acc.at[dibuf.at[kk]], add=True)

        def grp(g, _):
            pltpu.sync_copy(eidx_hbm.at[0, wid, pl.ds(g * G, G)], sibuf)
            pltpu.sync_copy(eidx_hbm.at[1, wid, pl.ds(g * G, G)], dibuf)
            gather(0, 0, gs0).start()

            def pair(p, _):
                k0 = 2 * p
                k1 = k0 + 1
                gather(k1, 1, gs1).start()
                gather(k0, 0, gs0).wait()
                scat(k0, 0)

                @pl.when(k1 + 1 < G)
                def _():
                    gather(k1 + 1, 0, gs0).start()

                gather(k1, 1, gs1).wait()
                scat(k1, 1)
                return 0

            lax.fori_loop(0, G // 2, pair, 0)
            return 0

        lax.fori_loop(0, nchunks_per_tile // G, grp, 0)

        plsc.subcore_barrier()

        def odesc(z):
            bz = sub + z * NS
            off = jnp.minimum(bz * ZR, n_acc - ZR)
            return pltpu.make_async_copy(
                acc.at[pl.ds(off, ZR), :], out_hbm.at[core, pl.ds(off, ZR), :],
                gs1,
            )

        no = (nob - sub + NS - 1) // NS
        lax.fori_loop(0, no, lambda z, _: (odesc(z).start(), 0)[1], 0)
        lax.fori_loop(0, no, lambda z, _: (odesc(z).wait(), 0)[1], 0)

    return segsum


def _make_gat_edges(n_rows, n_acc, nchunks_per_tile, ch):
    mesh = plsc.VectorSubcoreMesh(core_axis_name="c", subcore_axis_name="s")
    nzb = _ceil_div(n_acc, ch)
    nob = _ceil_div(n_acc, ZR)
    assert nchunks_per_tile % G == 0

    @functools.partial(
        pl.kernel,
        out_type=(
            jax.ShapeDtypeStruct((NC, n_acc, F), jnp.float32),
            jax.ShapeDtypeStruct((NC * n_acc,), jnp.float32),
        ),
        mesh=mesh,
        compiler_params=pltpu.CompilerParams(needs_layout_passes=False),
        scratch_types=[
            pltpu.VMEM((G, ch), jnp.int32),
            pltpu.VMEM((G, ch), jnp.int32),
            pltpu.VMEM((n_acc,), jnp.float32),
            pltpu.VMEM((n_acc,), jnp.float32),
            pltpu.VMEM((L,), jnp.float32),
            pltpu.VMEM((L,), jnp.float32),
            pltpu.VMEM((2, ch, F), jnp.float32),
            pltpu.VMEM((ch,), jnp.float32),
            pltpu.VMEM_SHARED((n_rows, F), jnp.float32),
            pltpu.VMEM_SHARED((n_rows,), jnp.float32),
            pltpu.SemaphoreType.DMA,
            pltpu.SemaphoreType.DMA,
        ],
    )
    def gat(h_hbm, s_hbm, t_hbm, smax_hbm, tmax_hbm, eidx_hbm,
            num_hbm, den_hbm, sibuf, dibuf, sv, tv, smv, tmv, rows, ee,
            accn, accd, gs0, gs1):
        core = lax.axis_index("c")
        sub = lax.axis_index("s")
        wid = sub * NC + core

        z16 = jnp.zeros((L,), jnp.float32)

        def zrow(i, _):
            for j in range(F // L):
                rows[0, i, pl.ds(j * L, L)] = z16
            return 0

        lax.fori_loop(0, ch, zrow, 0)
        for j in range(ch // L):
            ee[pl.ds(j * L, L)] = z16

        def zdescn(z):
            bz = sub + z * NS
            off = jnp.minimum(bz * ch, n_acc - ch)
            return pltpu.make_async_copy(rows.at[0], accn.at[pl.ds(off, ch), :], gs0)

        def zdescd(z):
            bz = sub + z * NS
            off = jnp.minimum(bz * ch, n_acc - ch)
            return pltpu.make_async_copy(ee, accd.at[pl.ds(off, ch)], gs1)

        nz = (nzb - sub + NS - 1) // NS
        lax.fori_loop(0, nz, lambda z, _: (zdescn(z).start(), zdescd(z).start(), 0)[2], 0)
        lax.fori_loop(0, nz, lambda z, _: (zdescn(z).wait(), zdescd(z).wait(), 0)[2], 0)

        if n_rows > n_acc:

            @pl.when(sub == 0)
            def _():
                pltpu.sync_copy(
                    rows.at[0, pl.ds(0, n_rows - n_acc), :],
                    accn.at[pl.ds(n_acc, n_rows - n_acc), :],
                )
                pltpu.sync_copy(
                    ee.at[pl.ds(0, n_rows - n_acc)],
                    accd.at[pl.ds(n_acc, n_rows - n_acc)],
                )

        pltpu.sync_copy(s_hbm, sv)
        pltpu.sync_copy(t_hbm, tv)
        pltpu.sync_copy(smax_hbm, smv)
        pltpu.sync_copy(tmax_hbm, tmv)

        plsc.subcore_barrier()

        cbound = jnp.maximum(smv[...] + tmv[...], 0.0)

        def gather(kk, b, sem):
            return pltpu.make_async_copy(
                h_hbm.at[sibuf.at[kk]], rows.at[b], sem
            )

        def process(kk, b):
            for j in range(ch // L):
                si = sibuf[kk, pl.ds(j * L, L)]
                di = dibuf[kk, pl.ds(j * L, L)]
                svv = plsc.load_gather(sv, [si])
                tvv = plsc.load_gather(tv, [di])
                e = svv + tvv
                e = jnp.maximum(e, 0.2 * e)
                ee[pl.ds(j * L, L)] = jnp.exp(e - cbound)

            def scale(i, _):
                w = plsc.load_gather(ee, [jnp.full((L,), 0, jnp.int32) + i])
                for j in range(F // L):
                    rows[b, i, pl.ds(j * L, L)] = rows[b, i, pl.ds(j * L, L)] * w
                return 0

            lax.fori_loop(0, ch, scale, 0)
            pltpu.sync_copy(rows.at[b], accn.at[dibuf.at[kk]], add=True)
            pltpu.sync_copy(ee, accd.at[dibuf.at[kk]], add=True)

        def grp(g, _):
            pltpu.sync_copy(eidx_hbm.at[0, wid, pl.ds(g * G, G)], sibuf)
            pltpu.sync_copy(eidx_hbm.at[1, wid, pl.ds(g * G, G)], dibuf)
            gather(0, 0, gs0).start()

            def pair(p, _):
                k0 = 2 * p
                k1 = k0 + 1
                gather(k1, 1, gs1).start()
                gather(k0, 0, gs0).wait()
                process(k0, 0)

                @pl.when(k1 + 1 < G)
                def _():
                    gather(k1 + 1, 0, gs0).start()

                gather(k1, 1, gs1).wait()
                process(k1, 1)
                return 0

            lax.fori_loop(0, G // 2, pair, 0)
            return 0

        lax.fori_loop(0, nchunks_per_tile // G, grp, 0)

        plsc.subcore_barrier()

        def odescn(z):
            bz = sub + z * NS
            off = jnp.minimum(bz * ZR, n_acc - ZR)
            return pltpu.make_async_copy(
                accn.at[pl.ds(off, ZR), :], num_hbm.at[core, pl.ds(off, ZR), :],
                gs0,
            )

        def odescd1(z):
            bz = sub + z * NS
            off = jnp.minimum(bz * ch, n_acc - ch)
            return pltpu.make_async_copy(
                accd.at[pl.ds(off, ch)],
                rows.at[0, lax.rem(z, ch), pl.ds(0, ch)], gs1,
            )

        def odescd2(z):
            bz = sub + z * NS
            off = jnp.minimum(bz * ch, n_acc - ch)
            return pltpu.make_async_copy(
                rows.at[0, lax.rem(z, ch), pl.ds(0, ch)],
                den_hbm.at[pl.ds(core * n_acc + off, ch)], gs1,
            )

        no = (nob - sub + NS - 1) // NS
        lax.fori_loop(0, no, lambda z, _: (odescn(z).start(), 0)[1], 0)
        lax.fori_loop(0, nz, lambda z, _: (odescd1(z).start(), 0)[1], 0)
        lax.fori_loop(0, no, lambda z, _: (odescn(z).wait(), 0)[1], 0)
        lax.fori_loop(0, nz, lambda z, _: (odescd1(z).wait(), 0)[1], 0)
        lax.fori_loop(0, nz, lambda z, _: (odescd2(z).start(), 0)[1], 0)
        lax.fori_loop(0, nz, lambda z, _: (odescd2(z).wait(), 0)[1], 0)

    return gat


def _gin_mlp(x, agg, wq1, bq1, wq2, bq2, wd1, bd1, wd2, bd2, nq, outer_relu):
    n = x.shape[0]
    B = nq
    grid = _ceil_div(n, B)

    def body(x_ref, a_ref, wq1_ref, bq1_ref, wq2_ref, bq2_ref,
             wd1_ref, bd1_ref, wd2_ref, bd2_ref, o_ref):
        i = pl.program_id(0)
        isq = i == 0
        w1 = jnp.where(isq, wq1_ref[...], wd1_ref[...])
        b1 = jnp.where(isq, bq1_ref[...], bd1_ref[...])
        w2 = jnp.where(isq, wq2_ref[...], wd2_ref[...])
        b2 = jnp.where(isq, bq2_ref[...], bd2_ref[...])
        a = a_ref[...]
        xa = x_ref[...] + a[0] + a[1]
        h = jnp.maximum(
            jnp.dot(xa, w1, preferred_element_type=jnp.float32) + b1[None, :],
            0.0,
        )
        h = jnp.dot(h, w2, preferred_element_type=jnp.float32) + b2[None, :]
        if outer_relu:
            h = jnp.maximum(h, 0.0)
        o_ref[...] = h

    wspec = pl.BlockSpec((F, F), lambda i: (0, 0))
    bspec = pl.BlockSpec((F,), lambda i: (0,))
    return pl.pallas_call(
        body,
        grid=(grid,),
        in_specs=[
            pl.BlockSpec((B, F), lambda i: (i, 0)),
            pl.BlockSpec((NC, B, F), lambda i: (0, i, 0)),
            wspec, bspec, wspec, bspec, wspec, bspec, wspec, bspec,
        ],
        out_specs=pl.BlockSpec((B, F), lambda i: (i, 0)),
        out_shape=jax.ShapeDtypeStruct((n, F), jnp.float32),
    )(x, agg, wq1, bq1, wq2, bq2, wd1, bd1, wd2, bd2)


def _gat_pre(x, w, a_src, a_dst):
    n = x.shape[0]
    B = 1024
    grid = _ceil_div(n, B)
    neg = -3.0e38

    def body(x_ref, w_ref, as_ref, ad_ref, h_ref, s_ref, t_ref, sm_ref, tm_ref):
        i = pl.program_id(0)
        h = jnp.dot(x_ref[...], w_ref[...], preferred_element_type=jnp.float32)
        h_ref[...] = h
        s = jnp.dot(h, as_ref[...][:, None], preferred_element_type=jnp.float32)
        t = jnp.dot(h, ad_ref[...][:, None], preferred_element_type=jnp.float32)
        s_ref[...] = s
        t_ref[...] = t
        rows = i * B + lax.broadcasted_iota(jnp.int32, (B, 1), 0)
        valid = rows < n
        sm = jnp.max(jnp.where(valid, s, neg))
        tm = jnp.max(jnp.where(valid, t, neg))

        @pl.when(i == 0)
        def _():
            sm_ref[...] = jnp.full((L,), neg, jnp.float32)
            tm_ref[...] = jnp.full((L,), neg, jnp.float32)

        sm_ref[...] = jnp.maximum(sm_ref[...], sm)
        tm_ref[...] = jnp.maximum(tm_ref[...], tm)

    return pl.pallas_call(
        body,
        grid=(grid,),
        in_specs=[
            pl.BlockSpec((B, F), lambda i: (i, 0)),
            pl.BlockSpec((F, F), lambda i: (0, 0)),
            pl.BlockSpec((F,), lambda i: (0,)),
            pl.BlockSpec((F,), lambda i: (0,)),
        ],
        out_specs=[
            pl.BlockSpec((B, F), lambda i: (i, 0)),
            pl.BlockSpec((B, 1), lambda i: (i, 0)),
            pl.BlockSpec((B, 1), lambda i: (i, 0)),
            pl.BlockSpec((L,), lambda i: (0,)),
            pl.BlockSpec((L,), lambda i: (0,)),
        ],
        out_shape=[
            jax.ShapeDtypeStruct((n, F), jnp.float32),
            jax.ShapeDtypeStruct((n, 1), jnp.float32),
            jax.ShapeDtypeStruct((n, 1), jnp.float32),
            jax.ShapeDtypeStruct((L,), jnp.float32),
            jax.ShapeDtypeStruct((L,), jnp.float32),
        ],
    )(x, w, a_src, a_dst)


def _att_fin(nump, denp, b, nq, ntot):
    B = 1024
    grid = _ceil_div(ntot, B)

    def body(n_ref, d_ref, b_ref, att_ref, qs_ref, ds_ref):
        i = pl.program_id(0)
        nsum = n_ref[...][0] + n_ref[...][1]
        den = d_ref[...][0] + d_ref[...][1] + 1e-16
        att = nsum / den[:, None] + b_ref[...][None, :]
        att_ref[...] = att
        rows = i * B + lax.broadcasted_iota(jnp.int32, (B, 1), 0)
        attv = jnp.where(rows < ntot, att, 0.0)
        qm = rows < nq

        @pl.when(i == 0)
        def _():
            qs_ref[...] = jnp.zeros((1, F), jnp.float32)
            ds_ref[...] = jnp.zeros((1, F), jnp.float32)

        qs_ref[...] += jnp.where(qm, attv, 0.0).sum(axis=0, keepdims=True)
        ds_ref[...] += jnp.where(qm, 0.0, attv).sum(axis=0, keepdims=True)

    return pl.pallas_call(
        body,
        grid=(grid,),
        in_specs=[
            pl.BlockSpec((NC, B, F), lambda i: (0, i, 0)),
            pl.BlockSpec((NC, B), lambda i: (0, i)),
            pl.BlockSpec((F,), lambda i: (0,)),
        ],
        out_specs=[
            pl.BlockSpec((B, F), lambda i: (i, 0)),
            pl.BlockSpec((1, F), lambda i: (0, 0)),
            pl.BlockSpec((1, F), lambda i: (0, 0)),
        ],
        out_shape=[
            jax.ShapeDtypeStruct((ntot, F), jnp.float32),
            jax.ShapeDtypeStruct((1, F), jnp.float32),
            jax.ShapeDtypeStruct((1, F), jnp.float32),
        ],
    )(nump, denp, b)


def _gin_mlp_out(x, agg, wq1, bq1, wq2, bq2, wd1, bd1, wd2, bd2, att, nq, nd):
    n = x.shape[0]
    B = nq
    grid = _ceil_div(n, B)

    def body(x_ref, a_ref, wq1_ref, bq1_ref, wq2_ref, bq2_ref,
             wd1_ref, bd1_ref, wd2_ref, bd2_ref, att_ref,
             oq_ref, od_ref, sq_ref, sd_ref):
        i = pl.program_id(0)
        isq = i == 0
        w1 = jnp.where(isq, wq1_ref[...], wd1_ref[...])
        b1 = jnp.where(isq, bq1_ref[...], bd1_ref[...])
        w2 = jnp.where(isq, wq2_ref[...], wd2_ref[...])
        b2 = jnp.where(isq, bq2_ref[...], bd2_ref[...])
        a = a_ref[...]
        xa = x_ref[...] + a[0] + a[1]
        h = jnp.maximum(
            jnp.dot(xa, w1, preferred_element_type=jnp.float32) + b1[None, :],
            0.0,
        )
        h = jnp.dot(h, w2, preferred_element_type=jnp.float32) + b2[None, :]
        att = att_ref[...]

        @pl.when(isq)
        def _():
            oq_ref[:, 0:F] = h
            oq_ref[:, F:2 * F] = att
            sq_ref[...] = h.sum(axis=0, keepdims=True)
            sd_ref[...] = jnp.zeros((1, F), jnp.float32)

        @pl.when(jnp.logical_not(isq))
        def _():
            od_ref[:, 0:F] = h
            od_ref[:, F:2 * F] = att
            rows = i * B + lax.broadcasted_iota(jnp.int32, (B, 1), 0)
            sd_ref[...] += jnp.where(rows < n, h, 0.0).sum(axis=0, keepdims=True)

    return pl.pallas_call(
        body,
        grid=(grid,),
        in_specs=[
            pl.BlockSpec((B, F), lambda i: (i, 0)),
            pl.BlockSpec((NC, B, F), lambda i: (0, i, 0)),
            pl.BlockSpec((F, F), lambda i: (0, 0)),
            pl.BlockSpec((F,), lambda i: (0,)),
            pl.BlockSpec((F, F), lambda i: (0, 0)),
            pl.BlockSpec((F,), lambda i: (0,)),
            pl.BlockSpec((F, F), lambda i: (0, 0)),
            pl.BlockSpec((F,), lambda i: (0,)),
            pl.BlockSpec((F, F), lambda i: (0, 0)),
            pl.BlockSpec((F,), lambda i: (0,)),
            pl.BlockSpec((B, F), lambda i: (i, 0)),
        ],
        out_specs=[
            pl.BlockSpec((B, 2 * F), lambda i: (0, 0)),
            pl.BlockSpec((B, 2 * F), lambda i: (jnp.maximum(i - 1, 0), 0)),
            pl.BlockSpec((1, F), lambda i: (0, 0)),
            pl.BlockSpec((1, F), lambda i: (0, 0)),
        ],
        out_shape=[
            jax.ShapeDtypeStruct((nq, 2 * F), jnp.float32),
            jax.ShapeDtypeStruct((nd, 2 * F), jnp.float32),
            jax.ShapeDtypeStruct((1, F), jnp.float32),
            jax.ShapeDtypeStruct((1, F), jnp.float32),
        ],
    )(x, agg, wq1, bq1, wq2, bq2, wd1, bd1, wd2, bd2, att)


def _head(qa, qb, da, db, w1, b1, w2, b2, w3, b3, w4, b4):
    def body(qa_ref, qb_ref, da_ref, db_ref, w1_ref, b1_ref, w2_ref, b2_ref,
             w3_ref, b3_ref, w4_ref, b4_ref, o_ref):
        w1v = w1_ref[...]
        h = (
            jnp.dot(qa_ref[...], w1v[0:128], preferred_element_type=jnp.float32)
            + jnp.dot(qb_ref[...], w1v[128:256], preferred_element_type=jnp.float32)
            + jnp.dot(da_ref[...], w1v[256:384], preferred_element_type=jnp.float32)
            + jnp.dot(db_ref[...], w1v[384:512], preferred_element_type=jnp.float32)
            + b1_ref[...][None, :]
        )
        h = jnp.dot(h, w2_ref[...], preferred_element_type=jnp.float32) + b2_ref[...][None, :]
        h = jnp.maximum(h, 0.0)
        h = jnp.dot(h, w3_ref[...], preferred_element_type=jnp.float32) + b3_ref[...][None, :]
        h = jnp.maximum(h, 0.0)
        h = jnp.dot(h, w4_ref[...], preferred_element_type=jnp.float32) + b4_ref[...][None, :]
        o_ref[...] = jnp.maximum(h, 0.0)

    return pl.pallas_call(
        body,
        out_shape=jax.ShapeDtypeStruct((1, 1), jnp.float32),
    )(qa, qb, da, db, w1, b1, w2, b2, w3, b3, w4, b4)


def _prep_edges(edge2, n_acc, ch):
    e = edge2.shape[1]
    unit = 2 * ch * NW
    epad = _ceil_div(e, unit) * unit
    npad = epad - e
    if npad:
        ar = jnp.arange(npad, dtype=jnp.int32)
        pad2 = jnp.stack([ar % 64, n_acc + ar % 8], axis=0)
        edge2 = jnp.concatenate([edge2, pad2], axis=1)
    per = epad // (NW * ch)
    eidx = edge2.reshape(2, NW, per, ch)
    return eidx, per, npad


def kernel(query_in_feat, data_in_feat, query_edge_list, data_edge_list,
           query2data_edge_list, qg_W1, qg_b1, qg_W2, qg_b2, qg_W3, qg_b3,
           qg_W4, qg_b4, dg_W1, dg_b1, dg_W2, dg_b2, dg_W3, dg_b3, dg_W4,
           dg_b4, gat_W, gat_a_src, gat_a_dst, gat_b, L1_W, L1_b, L2_W, L2_b,
           L3_W, L3_b, L4_W, L4_b):
    nq = query_in_feat.shape[0]
    nd = data_in_feat.shape[0]
    ntot = nq + nd

    qe = query_edge_list.astype(jnp.int32)
    de = data_edge_list.astype(jnp.int32)
    xe = query2data_edge_list.astype(jnp.int32)

    cedge = jnp.concatenate([qe, de + nq], axis=1)
    cidx, cper, cpad = _prep_edges(cedge, ntot, CH_GIN)
    xidx, xper, xpad = _prep_edges(xe, ntot, CH_GAT)

    segsum = _make_segsum(ntot + (8 if cpad else 0), ntot, cper, CH_GIN)
    gat_edges = _make_gat_edges(ntot + (8 if xpad else 0), ntot, xper, CH_GAT)

    x0 = jnp.concatenate([query_in_feat, data_in_feat], axis=0)

    agg1 = segsum(x0, cidx)
    h1 = _gin_mlp(x0, agg1, qg_W1, qg_b1, qg_W2, qg_b2,
                  dg_W1, dg_b1, dg_W2, dg_b2, nq, outer_relu=True)

    hg, s, t, smax, tmax = _gat_pre(x0, gat_W, gat_a_src, gat_a_dst)
    nump, denp = gat_edges(hg, s[:, 0], t[:, 0], smax, tmax, xidx)
    denp2 = denp.reshape(NC, ntot)
    att, qsB, dsB = _att_fin(nump, denp2, gat_b, nq, ntot)

    agg2 = segsum(h1, cidx)
    out_q, out_d, qsA, dsA = _gin_mlp_out(
        h1, agg2, qg_W3, qg_b3, qg_W4, qg_b4,
        dg_W3, dg_b3, dg_W4, dg_b4, att, nq, nd)

    pred = _head(qsA, qsB, dsA, dsB, L1_W, L1_b, L2_W, L2_b, L3_W, L3_b,
                 L4_W, L4_b)

    return (pred, out_q, out_d)
